# baseline (speedup 1.0000x reference)
import functools

import jax
import jax.numpy as jnp
import numpy as np
from jax import lax
from jax.experimental import pallas as pl
from jax.experimental.pallas import tpu as pltpu
from jax.experimental.pallas import tpu_sc as plsc

_N = 10000
_NP = 10240
_M = 128
_E = 160000
_EPAD = 161024
_KAPPA = 0.9
_NITER = 15
_PITERS = 50
_NC = 2
_NS = 16
_NW = _NC * _NS
_EPT = _E // _NW
_CH = 128
_NCHS = 40
_EPAD2 = _NW * _NCHS * _CH
_MVCH = 512
_NCHM = 10
_ROWS_T = _NP // _NS

_mesh = plsc.VectorSubcoreMesh(core_axis_name="c", subcore_axis_name="s")


def _spmm_body(zt, rs, cs, ws, out, yt_sh, rs_b, cs_b, ws_b, rowbuf, sem):
    cid = lax.axis_index("c")
    sid = lax.axis_index("s")
    wid = sid * _NC + cid
    zero = jnp.zeros((16,), jnp.float32)

    def zb(i, _):
        for f in range(8):
            rowbuf[i, pl.ds(f * 16, 16)] = zero
        return 0

    lax.fori_loop(0, _CH, zb, 0)
    for k in range(_ROWS_T // _CH):
        pltpu.sync_copy(rowbuf, yt_sh.at[pl.ds(sid * _ROWS_T + k * _CH, _CH)])
    plsc.subcore_barrier()

    t_lo = wid * _EPT
    t_hi = t_lo + _EPT
    lane = lax.broadcasted_iota(jnp.int32, (16,), 0)

    def chunk(k, _):
        base = t_lo + k * _CH
        pltpu.sync_copy(rs.at[pl.ds(base, _CH)], rs_b)
        pltpu.sync_copy(cs.at[pl.ds(base, _CH)], cs_b)
        pltpu.sync_copy(ws.at[pl.ds(base, _CH)], ws_b)
        pltpu.async_copy(zt.at[rs_b], rowbuf, sem).wait()

        def grp16(g, _):
            off = g * 16
            w16 = ws_b[pl.ds(off, 16)]
            wv16 = jnp.where(base + off + lane < t_hi, w16, 0.0)
            for j in range(16):
                wb = jnp.full((16,), wv16[j], jnp.float32)
                for f in range(8):
                    x = rowbuf[off + j, pl.ds(f * 16, 16)]
                    rowbuf[off + j, pl.ds(f * 16, 16)] = x * wb
            return 0

        lax.fori_loop(0, _CH // 16, grp16, 0, unroll=2)
        pltpu.sync_copy(rowbuf, yt_sh.at[cs_b], add=True)
        return 0

    lax.fori_loop(0, _NCHS, chunk, 0)
    plsc.subcore_barrier()
    pltpu.sync_copy(yt_sh.at[pl.ds(sid * _ROWS_T, _ROWS_T)],
                    out.at[cid, pl.ds(sid * _ROWS_T, _ROWS_T)])


def _spmm(zt, rs, cs, ws):
    return pl.kernel(
        _spmm_body,
        out_type=jax.ShapeDtypeStruct((_NC, _NP, _M), jnp.float32),
        mesh=_mesh,
        scratch_types=[
            pltpu.VMEM_SHARED((_NP, _M), jnp.float32),
            pltpu.VMEM((_CH,), jnp.int32),
            pltpu.VMEM((_CH,), jnp.int32),
            pltpu.VMEM((_CH,), jnp.float32),
            pltpu.VMEM((_CH, _M), jnp.float32),
            pltpu.SemaphoreType.DMA,
        ],
        compiler_params=pltpu.CompilerParams(needs_layout_passes=False, use_tc_tiling_on_sc=False),
    )(zt, rs, cs, ws)


def _mv_body(up, rr, cc, ww, idx5, upn,
             av_sh, up_loc, u_loc, av_loc, zb40, r_b, c_b, w_b, idx_loc,
             red16, sem):
    cid = lax.axis_index("c")
    sid = lax.axis_index("s")
    wid = sid * _NC + cid
    lane = lax.broadcasted_iota(jnp.int32, (16,), 0)
    pltpu.sync_copy(up, up_loc)
    pltpu.sync_copy(idx5, idx_loc)

    def addu(i, acc):
        u = up_loc[0, i] + up_loc[1, i]
        u_loc[i] = u
        return acc + u * u

    acc = lax.fori_loop(0, _NP // 16, addu, jnp.zeros((16,), jnp.float32),
                        unroll=4)
    dnums = lax.GatherDimensionNumbers(
        offset_dims=(), collapsed_slice_dims=(0,), start_index_map=(0,))
    for sh in (8, 4, 2, 1):
        perm = jnp.bitwise_xor(lane, sh)[:, None]
        acc = acc + lax.gather(acc, perm, dnums, (1,),
                               mode=lax.GatherScatterMode.PROMISE_IN_BOUNDS)
    s2v = acc
    yi = jnp.int32(0x5F3759DF) - lax.shift_right_logical(
        lax.bitcast_convert_type(s2v, jnp.int32), 1)
    y = lax.bitcast_convert_type(yi, jnp.float32)
    for _ in range(3):
        y = y * (1.5 - 0.5 * s2v * y * y)
    inv_v = y

    zero = jnp.zeros((16,), jnp.float32)

    def zav(i, _):
        av_loc[i] = zero
        return 0

    lax.fori_loop(0, _NP // 16, zav, 0)

    def zzb(i, _):
        zb40[i] = zero
        return 0

    lax.fori_loop(0, _ROWS_T // 16, zzb, 0)
    pltpu.sync_copy(zb40, av_sh.at[pl.ds(sid * (_ROWS_T // 16), _ROWS_T // 16)])
    plsc.subcore_barrier()

    t_lo = wid * _EPT
    pltpu.sync_copy(rr.at[pl.ds(t_lo, _MVCH * _NCHM)], r_b)
    pltpu.sync_copy(cc.at[pl.ds(t_lo, _MVCH * _NCHM)], c_b)
    pltpu.sync_copy(ww.at[pl.ds(t_lo, _MVCH * _NCHM)], w_b)

    def grp(j, _):
        off = j * 16
        c16 = c_b[pl.ds(off, 16)]
        w16 = w_b[pl.ds(off, 16)]
        r16 = r_b[pl.ds(off, 16)]
        g = plsc.load_gather(
            u_loc, [lax.shift_right_logical(c16, 4),
                    jnp.bitwise_and(c16, 15)])
        wv = jnp.where(off + lane < _EPT, w16, 0.0)
        val = g * wv * inv_v
        plsc.addupdate_scatter(
            av_loc, [lax.shift_right_logical(r16, 4),
                     jnp.bitwise_and(r16, 15)], val)
        return 0

    lax.fori_loop(0, _MVCH * _NCHM // 16, grp, 0, unroll=2)
    for k in range(5):
        pltpu.sync_copy(av_loc.at[pl.ds(k * 128, 128)],
                        av_sh.at[idx_loc.at[k]], add=True)
    plsc.subcore_barrier()
    pltpu.sync_copy(av_sh.at[pl.ds(sid * (_ROWS_T // 16), _ROWS_T // 16)],
                    upn.at[cid, pl.ds(sid * (_ROWS_T // 16), _ROWS_T // 16)])


def _matvec(up, rr, cc, ww, idx5):
    return pl.kernel(
        _mv_body,
        out_type=jax.ShapeDtypeStruct((_NC, _NP // 16, 16), jnp.float32),
        mesh=_mesh,
        scratch_types=[
            pltpu.VMEM_SHARED((_NP // 16, 16), jnp.float32),
            pltpu.VMEM((_NC, _NP // 16, 16), jnp.float32),
            pltpu.VMEM((_NP // 16, 16), jnp.float32),
            pltpu.VMEM((_NP // 16, 16), jnp.float32),
            pltpu.VMEM((_ROWS_T // 16, 16), jnp.float32),
            pltpu.VMEM((_MVCH * _NCHM,), jnp.int32),
            pltpu.VMEM((_MVCH * _NCHM,), jnp.int32),
            pltpu.VMEM((_MVCH * _NCHM,), jnp.float32),
            pltpu.VMEM((5, 128), jnp.int32),
            pltpu.VMEM((16,), jnp.float32),
            pltpu.SemaphoreType.DMA,
        ],
        compiler_params=pltpu.CompilerParams(needs_layout_passes=False, use_tc_tiling_on_sc=False),
    )(up, rr, cc, ww, idx5)


_BN = 1280


def _omega_body(f_ref, om_ref, o_ref):
    o_ref[...] = lax.dot_general(
        f_ref[...], om_ref[...], (((0,), (1,)), ((), ())),
        preferred_element_type=jnp.float32)


def _omega(fpad, Om):
    return pl.pallas_call(
        _omega_body,
        grid=(_NP // _BN,),
        in_specs=[pl.BlockSpec((256, _BN), lambda i: (0, i)),
                  pl.BlockSpec((_M, 256), lambda i: (0, 0))],
        out_specs=pl.BlockSpec((_BN, _M), lambda i: (i, 0)),
        out_shape=jax.ShapeDtypeStruct((_NP, _M), jnp.float32),
    )(fpad, Om)


def _pairsum_body(y_ref, o_ref):
    o_ref[...] = y_ref[0] + y_ref[1]


def _pairsum(yp):
    return pl.pallas_call(
        _pairsum_body,
        grid=(_NP // _BN,),
        in_specs=[pl.BlockSpec((2, _BN, _M), lambda i: (0, i, 0))],
        out_specs=pl.BlockSpec((_BN, _M), lambda i: (i, 0)),
        out_shape=jax.ShapeDtypeStruct((_NP, _M), jnp.float32),
    )(yp)


def _first_body(b_ref, w_ref, o_ref):
    xn = jnp.maximum(b_ref[...], 0.0)
    o_ref[...] = lax.dot_general(
        xn, w_ref[...], (((1,), (1,)), ((), ())),
        preferred_element_type=jnp.float32)


def _first(bT, Wp):
    return pl.pallas_call(
        _first_body,
        grid=(_NP // _BN,),
        in_specs=[pl.BlockSpec((_BN, _M), lambda i: (i, 0)),
                  pl.BlockSpec((_M, _M), lambda i: (0, 0))],
        out_specs=pl.BlockSpec((_BN, _M), lambda i: (i, 0)),
        out_shape=jax.ShapeDtypeStruct((_NP, _M), jnp.float32),
    )(bT, Wp)


def _step_body(y_ref, b_ref, w_ref, o_ref):
    xn = jnp.maximum(y_ref[0] + y_ref[1] + b_ref[...], 0.0)
    o_ref[...] = lax.dot_general(
        xn, w_ref[...], (((1,), (1,)), ((), ())),
        preferred_element_type=jnp.float32)


def _step(yp, bT, Wp):
    return pl.pallas_call(
        _step_body,
        grid=(_NP // _BN,),
        in_specs=[pl.BlockSpec((2, _BN, _M), lambda i: (0, i, 0)),
                  pl.BlockSpec((_BN, _M), lambda i: (i, 0)),
                  pl.BlockSpec((_M, _M), lambda i: (0, 0))],
        out_specs=pl.BlockSpec((_BN, _M), lambda i: (i, 0)),
        out_shape=jax.ShapeDtypeStruct((_NP, _M), jnp.float32),
    )(yp, bT, Wp)


def _final_body(y_ref, b_ref, v_ref, o_ref):
    xn = jnp.maximum(y_ref[0] + y_ref[1] + b_ref[...], 0.0)
    s = jnp.sum(xn * xn, axis=1, keepdims=True)
    nrm = jnp.maximum(jnp.sqrt(s), 1e-12)
    o_ref[...] = lax.dot_general(
        xn / nrm, v_ref[...], (((1,), (1,)), ((), ())),
        preferred_element_type=jnp.float32)


def _final(yp, bT, Vw):
    return pl.pallas_call(
        _final_body,
        grid=(_NP // _BN,),
        in_specs=[pl.BlockSpec((2, _BN, _M), lambda i: (0, i, 0)),
                  pl.BlockSpec((_BN, _M), lambda i: (i, 0)),
                  pl.BlockSpec((64, _M), lambda i: (0, 0))],
        out_specs=pl.BlockSpec((_BN, 64), lambda i: (i, 0)),
        out_shape=jax.ShapeDtypeStruct((_NP, 64), jnp.float32),
    )(yp, bT, Vw)


def _proj_body(w_ref, u_ref, o_ref):
    u = u_ref[0:1, :] + u_ref[1:2, :]
    s2 = jnp.sum(u * u)
    arho = jnp.sqrt(s2) + 1e-12
    v = _KAPPA / arho
    A = w_ref[...]
    absA = jnp.abs(A)
    row_sum = jnp.sum(absA, axis=1, keepdims=True)
    hi0 = jnp.max(absA, axis=1, keepdims=True)
    lo0 = jnp.zeros_like(hi0)

    def bis(_, carry):
        lo, hi = carry
        mid = 0.5 * (lo + hi)
        f = jnp.sum(jnp.maximum(absA - mid, 0.0), axis=1, keepdims=True)
        gt = f > v
        return jnp.where(gt, mid, lo), jnp.where(gt, hi, mid)

    lo, hi = lax.fori_loop(0, 60, bis, (lo0, hi0))
    theta = 0.5 * (lo + hi)
    proj = jnp.maximum(absA - theta, 0.0) * jnp.sign(A)
    o_ref[...] = jnp.where(row_sum > v, proj, A)


def _proj(W, up50):
    return pl.pallas_call(
        _proj_body,
        in_specs=[pl.BlockSpec((_M, _M), lambda: (0, 0)),
                  pl.BlockSpec((_NC, _NP), lambda: (0, 0))],
        out_specs=pl.BlockSpec((_M, _M), lambda: (0, 0)),
        out_shape=jax.ShapeDtypeStruct((_M, _M), jnp.float32),
    )(W, up50)


def kernel(features, edge_index, edge_weight, W, Omega_1, V_w):
    f32 = jnp.float32
    row = edge_index[0]
    col = edge_index[1]
    rp = jnp.pad(row, (0, _EPAD - _E))
    cp = jnp.pad(col, (0, _EPAD - _E))
    wp = jnp.pad(edge_weight, (0, _EPAD - _E))
    fpad = jnp.pad(features, ((0, 0), (0, _NP - _N)))
    idx5 = jnp.arange(_NP // 16, dtype=jnp.int32).reshape(5, 128)

    zt0 = _omega(fpad, Omega_1)
    bpair = _spmm(zt0, rp, cp, wp)
    bT = _pairsum(bpair)

    v0 = jnp.concatenate([jnp.full((_N,), 1.0 / np.sqrt(_N), f32),
                          jnp.zeros((_NP - _N,), f32)])
    up0 = jnp.stack([v0, jnp.zeros((_NP,), f32)]).reshape(_NC, _NP // 16, 16)

    def mvstep(up, _):
        return _matvec(up, rp, cp, wp, idx5), None

    up50, _ = lax.scan(mvstep, up0, None, length=_PITERS)
    Wp = _proj(W, up50.reshape(_NC, _NP))

    zt = _first(bT, Wp)

    def fpstep(zt, _):
        return _step(_spmm(zt, rp, cp, wp), bT, Wp), None

    zt, _ = lax.scan(fpstep, zt, None, length=_NITER - 2)
    outp = _final(_spmm(zt, rp, cp, wp), bT, V_w)
    return outp[:_N]


# speedup vs baseline: 1.0044x; 1.0044x over previous
import functools

import jax
import jax.numpy as jnp
import numpy as np
from jax import lax
from jax.experimental import pallas as pl
from jax.experimental.pallas import tpu as pltpu
from jax.experimental.pallas import tpu_sc as plsc

_N = 10000
_NP = 10240
_M = 128
_E = 160000
_EPAD = 161024
_KAPPA = 0.9
_NITER = 15
_PITERS = 50
_NC = 2
_NS = 16
_NW = _NC * _NS
_EPT = _E // _NW
_CH = 128
_NCHS = 40
_EPAD2 = _NW * _NCHS * _CH
_MVCH = 512
_NCHM = 10
_ROWS_T = _NP // _NS

_mesh = plsc.VectorSubcoreMesh(core_axis_name="c", subcore_axis_name="s")


def _spmm_body(zt, rs, cs, ws, out, yt_sh, rs_b, cs_b, ws_b, rowbuf, sem):
    cid = lax.axis_index("c")
    sid = lax.axis_index("s")
    wid = sid * _NC + cid
    zero = jnp.zeros((16,), jnp.float32)

    def zb(i, _):
        for f in range(8):
            rowbuf[i, pl.ds(f * 16, 16)] = zero
        return 0

    lax.fori_loop(0, _CH, zb, 0)
    for k in range(_ROWS_T // _CH):
        pltpu.sync_copy(rowbuf, yt_sh.at[pl.ds(sid * _ROWS_T + k * _CH, _CH)])
    plsc.subcore_barrier()

    t_lo = wid * _EPT
    t_hi = t_lo + _EPT
    lane = lax.broadcasted_iota(jnp.int32, (16,), 0)

    def chunk(k, _):
        base = t_lo + k * _CH
        pltpu.sync_copy(rs.at[pl.ds(base, _CH)], rs_b)
        pltpu.sync_copy(cs.at[pl.ds(base, _CH)], cs_b)
        pltpu.sync_copy(ws.at[pl.ds(base, _CH)], ws_b)
        pltpu.async_copy(zt.at[rs_b], rowbuf, sem).wait()

        def grp16(g, _):
            off = g * 16
            w16 = ws_b[pl.ds(off, 16)]
            wv16 = jnp.where(base + off + lane < t_hi, w16, 0.0)
            for j in range(16):
                wb = jnp.full((16,), wv16[j], jnp.float32)
                for f in range(8):
                    x = rowbuf[off + j, pl.ds(f * 16, 16)]
                    rowbuf[off + j, pl.ds(f * 16, 16)] = x * wb
            return 0

        lax.fori_loop(0, _CH // 16, grp16, 0)
        pltpu.sync_copy(rowbuf, yt_sh.at[cs_b], add=True)
        return 0

    lax.fori_loop(0, _NCHS, chunk, 0)
    plsc.subcore_barrier()
    pltpu.sync_copy(yt_sh.at[pl.ds(sid * _ROWS_T, _ROWS_T)],
                    out.at[cid, pl.ds(sid * _ROWS_T, _ROWS_T)])


def _spmm(zt, rs, cs, ws):
    return pl.kernel(
        _spmm_body,
        out_type=jax.ShapeDtypeStruct((_NC, _NP, _M), jnp.float32),
        mesh=_mesh,
        scratch_types=[
            pltpu.VMEM_SHARED((_NP, _M), jnp.float32),
            pltpu.VMEM((_CH,), jnp.int32),
            pltpu.VMEM((_CH,), jnp.int32),
            pltpu.VMEM((_CH,), jnp.float32),
            pltpu.VMEM((_CH, _M), jnp.float32),
            pltpu.SemaphoreType.DMA,
        ],
        compiler_params=pltpu.CompilerParams(needs_layout_passes=False, use_tc_tiling_on_sc=False),
    )(zt, rs, cs, ws)


def _mv_body(up, rr, cc, ww, idx5, upn,
             av_sh, up_loc, u_loc, av_loc, zb40, r_b, c_b, w_b, idx_loc,
             red16, sem):
    cid = lax.axis_index("c")
    sid = lax.axis_index("s")
    wid = sid * _NC + cid
    lane = lax.broadcasted_iota(jnp.int32, (16,), 0)
    pltpu.sync_copy(up, up_loc)
    pltpu.sync_copy(idx5, idx_loc)

    def addu(i, acc):
        u = up_loc[0, i] + up_loc[1, i]
        u_loc[i] = u
        return acc + u * u

    acc = lax.fori_loop(0, _NP // 16, addu, jnp.zeros((16,), jnp.float32),
                        unroll=4)
    dnums = lax.GatherDimensionNumbers(
        offset_dims=(), collapsed_slice_dims=(0,), start_index_map=(0,))
    for sh in (8, 4, 2, 1):
        perm = jnp.bitwise_xor(lane, sh)[:, None]
        acc = acc + lax.gather(acc, perm, dnums, (1,),
                               mode=lax.GatherScatterMode.PROMISE_IN_BOUNDS)
    s2v = acc
    yi = jnp.int32(0x5F3759DF) - lax.shift_right_logical(
        lax.bitcast_convert_type(s2v, jnp.int32), 1)
    y = lax.bitcast_convert_type(yi, jnp.float32)
    for _ in range(3):
        y = y * (1.5 - 0.5 * s2v * y * y)
    inv_v = y

    zero = jnp.zeros((16,), jnp.float32)

    def zav(i, _):
        av_loc[i] = zero
        return 0

    lax.fori_loop(0, _NP // 16, zav, 0)

    def zzb(i, _):
        zb40[i] = zero
        return 0

    lax.fori_loop(0, _ROWS_T // 16, zzb, 0)
    pltpu.sync_copy(zb40, av_sh.at[pl.ds(sid * (_ROWS_T // 16), _ROWS_T // 16)])
    plsc.subcore_barrier()

    t_lo = wid * _EPT
    pltpu.sync_copy(rr.at[pl.ds(t_lo, _MVCH * _NCHM)], r_b)
    pltpu.sync_copy(cc.at[pl.ds(t_lo, _MVCH * _NCHM)], c_b)
    pltpu.sync_copy(ww.at[pl.ds(t_lo, _MVCH * _NCHM)], w_b)

    def grp(j, _):
        off = j * 16
        c16 = c_b[pl.ds(off, 16)]
        w16 = w_b[pl.ds(off, 16)]
        r16 = r_b[pl.ds(off, 16)]
        g = plsc.load_gather(
            u_loc, [lax.shift_right_logical(c16, 4),
                    jnp.bitwise_and(c16, 15)])
        wv = jnp.where(off + lane < _EPT, w16, 0.0)
        val = g * wv * inv_v
        plsc.addupdate_scatter(
            av_loc, [lax.shift_right_logical(r16, 4),
                     jnp.bitwise_and(r16, 15)], val)
        return 0

    lax.fori_loop(0, _MVCH * _NCHM // 16, grp, 0, unroll=2)
    for k in range(5):
        pltpu.sync_copy(av_loc.at[pl.ds(k * 128, 128)],
                        av_sh.at[idx_loc.at[k]], add=True)
    plsc.subcore_barrier()
    pltpu.sync_copy(av_sh.at[pl.ds(sid * (_ROWS_T // 16), _ROWS_T // 16)],
                    upn.at[cid, pl.ds(sid * (_ROWS_T // 16), _ROWS_T // 16)])


def _matvec(up, rr, cc, ww, idx5):
    return pl.kernel(
        _mv_body,
        out_type=jax.ShapeDtypeStruct((_NC, _NP // 16, 16), jnp.float32),
        mesh=_mesh,
        scratch_types=[
            pltpu.VMEM_SHARED((_NP // 16, 16), jnp.float32),
            pltpu.VMEM((_NC, _NP // 16, 16), jnp.float32),
            pltpu.VMEM((_NP // 16, 16), jnp.float32),
            pltpu.VMEM((_NP // 16, 16), jnp.float32),
            pltpu.VMEM((_ROWS_T // 16, 16), jnp.float32),
            pltpu.VMEM((_MVCH * _NCHM,), jnp.int32),
            pltpu.VMEM((_MVCH * _NCHM,), jnp.int32),
            pltpu.VMEM((_MVCH * _NCHM,), jnp.float32),
            pltpu.VMEM((5, 128), jnp.int32),
            pltpu.VMEM((16,), jnp.float32),
            pltpu.SemaphoreType.DMA,
        ],
        compiler_params=pltpu.CompilerParams(needs_layout_passes=False, use_tc_tiling_on_sc=False),
    )(up, rr, cc, ww, idx5)


_BN = 1280


def _omega_body(f_ref, om_ref, o_ref):
    o_ref[...] = lax.dot_general(
        f_ref[...], om_ref[...], (((0,), (1,)), ((), ())),
        preferred_element_type=jnp.float32)


def _omega(fpad, Om):
    return pl.pallas_call(
        _omega_body,
        grid=(_NP // _BN,),
        in_specs=[pl.BlockSpec((256, _BN), lambda i: (0, i)),
                  pl.BlockSpec((_M, 256), lambda i: (0, 0))],
        out_specs=pl.BlockSpec((_BN, _M), lambda i: (i, 0)),
        out_shape=jax.ShapeDtypeStruct((_NP, _M), jnp.float32),
    )(fpad, Om)


def _pairsum_body(y_ref, o_ref):
    o_ref[...] = y_ref[0] + y_ref[1]


def _pairsum(yp):
    return pl.pallas_call(
        _pairsum_body,
        grid=(_NP // _BN,),
        in_specs=[pl.BlockSpec((2, _BN, _M), lambda i: (0, i, 0))],
        out_specs=pl.BlockSpec((_BN, _M), lambda i: (i, 0)),
        out_shape=jax.ShapeDtypeStruct((_NP, _M), jnp.float32),
    )(yp)


def _first_body(b_ref, w_ref, o_ref):
    xn = jnp.maximum(b_ref[...], 0.0)
    o_ref[...] = lax.dot_general(
        xn, w_ref[...], (((1,), (0,)), ((), ())),
        preferred_element_type=jnp.float32)


def _first(bT, Wp):
    return pl.pallas_call(
        _first_body,
        grid=(_NP // _BN,),
        in_specs=[pl.BlockSpec((_BN, _M), lambda i: (i, 0)),
                  pl.BlockSpec((_M, _M), lambda i: (0, 0))],
        out_specs=pl.BlockSpec((_BN, _M), lambda i: (i, 0)),
        out_shape=jax.ShapeDtypeStruct((_NP, _M), jnp.float32),
    )(bT, Wp)


def _step_body(y_ref, b_ref, w_ref, o_ref):
    xn = jnp.maximum(y_ref[0] + y_ref[1] + b_ref[...], 0.0)
    o_ref[...] = lax.dot_general(
        xn, w_ref[...], (((1,), (0,)), ((), ())),
        preferred_element_type=jnp.float32)


def _step(yp, bT, Wp):
    return pl.pallas_call(
        _step_body,
        grid=(_NP // _BN,),
        in_specs=[pl.BlockSpec((2, _BN, _M), lambda i: (0, i, 0)),
                  pl.BlockSpec((_BN, _M), lambda i: (i, 0)),
                  pl.BlockSpec((_M, _M), lambda i: (0, 0))],
        out_specs=pl.BlockSpec((_BN, _M), lambda i: (i, 0)),
        out_shape=jax.ShapeDtypeStruct((_NP, _M), jnp.float32),
    )(yp, bT, Wp)


def _final_body(y_ref, b_ref, v_ref, o_ref):
    xn = jnp.maximum(y_ref[0] + y_ref[1] + b_ref[...], 0.0)
    s = jnp.sum(xn * xn, axis=1, keepdims=True)
    nrm = jnp.maximum(jnp.sqrt(s), 1e-12)
    o_ref[...] = lax.dot_general(
        xn / nrm, v_ref[...], (((1,), (1,)), ((), ())),
        preferred_element_type=jnp.float32)


def _final(yp, bT, Vw):
    return pl.pallas_call(
        _final_body,
        grid=(_NP // _BN,),
        in_specs=[pl.BlockSpec((2, _BN, _M), lambda i: (0, i, 0)),
                  pl.BlockSpec((_BN, _M), lambda i: (i, 0)),
                  pl.BlockSpec((64, _M), lambda i: (0, 0))],
        out_specs=pl.BlockSpec((_BN, 64), lambda i: (i, 0)),
        out_shape=jax.ShapeDtypeStruct((_NP, 64), jnp.float32),
    )(yp, bT, Vw)


def _proj_body(w_ref, u_ref, o_ref):
    u = u_ref[0:1, :] + u_ref[1:2, :]
    s2 = jnp.sum(u * u)
    arho = jnp.sqrt(s2) + 1e-12
    v = _KAPPA / arho
    A = w_ref[...]
    absA = jnp.abs(A)
    row_sum = jnp.sum(absA, axis=1, keepdims=True)
    hi0 = jnp.max(absA, axis=1, keepdims=True)
    lo0 = jnp.zeros_like(hi0)

    def bis(_, carry):
        lo, hi = carry
        mid = 0.5 * (lo + hi)
        f = jnp.sum(jnp.maximum(absA - mid, 0.0), axis=1, keepdims=True)
        gt = f > v
        return jnp.where(gt, mid, lo), jnp.where(gt, hi, mid)

    lo, hi = lax.fori_loop(0, 60, bis, (lo0, hi0))
    theta = 0.5 * (lo + hi)
    proj = jnp.maximum(absA - theta, 0.0) * jnp.sign(A)
    o_ref[...] = jnp.where(row_sum > v, proj, A)


def _proj(W, up50):
    return pl.pallas_call(
        _proj_body,
        in_specs=[pl.BlockSpec((_M, _M), lambda: (0, 0)),
                  pl.BlockSpec((_NC, _NP), lambda: (0, 0))],
        out_specs=pl.BlockSpec((_M, _M), lambda: (0, 0)),
        out_shape=jax.ShapeDtypeStruct((_M, _M), jnp.float32),
    )(W, up50)


def kernel(features, edge_index, edge_weight, W, Omega_1, V_w):
    f32 = jnp.float32
    row = edge_index[0]
    col = edge_index[1]
    rp = jnp.pad(row, (0, _EPAD - _E))
    cp = jnp.pad(col, (0, _EPAD - _E))
    wp = jnp.pad(edge_weight, (0, _EPAD - _E))
    fpad = jnp.pad(features, ((0, 0), (0, _NP - _N)))
    idx5 = jnp.arange(_NP // 16, dtype=jnp.int32).reshape(5, 128)

    zt0 = _omega(fpad, Omega_1)
    bpair = _spmm(zt0, rp, cp, wp)
    bT = _pairsum(bpair)

    v0 = jnp.concatenate([jnp.full((_N,), 1.0 / np.sqrt(_N), f32),
                          jnp.zeros((_NP - _N,), f32)])
    up0 = jnp.stack([v0, jnp.zeros((_NP,), f32)]).reshape(_NC, _NP // 16, 16)

    def mvstep(up, _):
        return _matvec(up, rp, cp, wp, idx5), None

    up50, _ = lax.scan(mvstep, up0, None, length=_PITERS)
    Wp = _proj(W, up50.reshape(_NC, _NP))

    WpT = Wp.T
    zt = _first(bT, WpT)

    def fpstep(zt, _):
        return _step(_spmm(zt, rp, cp, wp), bT, WpT), None

    zt, _ = lax.scan(fpstep, zt, None, length=_NITER - 2)
    outp = _final(_spmm(zt, rp, cp, wp), bT, V_w)
    return outp[:_N]


# speedup vs baseline: 1.2332x; 1.2278x over previous
import functools

import jax
import jax.numpy as jnp
import numpy as np
from jax import lax
from jax.experimental import pallas as pl
from jax.experimental.pallas import tpu as pltpu
from jax.experimental.pallas import tpu_sc as plsc

_N = 10000
_NP = 10240
_M = 128
_E = 160000
_EPAD = 161024
_KAPPA = 0.9
_NITER = 15
_PITERS = 50
_NC = 2
_NS = 16
_NW = _NC * _NS
_EPT = _E // _NW
_CH = 128
_NCHS = 40
_EPAD2 = _NW * _NCHS * _CH
_MVCH = 512
_NCHM = 10
_ROWS_T = _NP // _NS

_mesh = plsc.VectorSubcoreMesh(core_axis_name="c", subcore_axis_name="s")


def _spmm_body(zt, rs, cs, ws, out, yt_sh, rs_b, cs_b, ws_b, rowbuf,
               rs_b2, cs_b2, ws_b2, rowbuf2, sem, sem2):
    cid = lax.axis_index("c")
    sid = lax.axis_index("s")
    wid = sid * _NC + cid
    zero = jnp.zeros((16,), jnp.float32)

    def zb(i, _):
        for f in range(8):
            rowbuf[i, pl.ds(f * 16, 16)] = zero
        return 0

    lax.fori_loop(0, _CH, zb, 0)
    for k in range(_ROWS_T // _CH):
        pltpu.sync_copy(rowbuf, yt_sh.at[pl.ds(sid * _ROWS_T + k * _CH, _CH)])
    plsc.subcore_barrier()

    t_lo = wid * _EPT
    t_hi = t_lo + _EPT
    lane = lax.broadcasted_iota(jnp.int32, (16,), 0)

    def scale_scatter(buf, wbuf, cbuf, base):
        def grp16(g, _):
            off = g * 16
            w16 = wbuf[pl.ds(off, 16)]
            wv16 = jnp.where(base + off + lane < t_hi, w16, 0.0)
            for j in range(16):
                wb = jnp.full((16,), wv16[j], jnp.float32)
                for f in range(8):
                    x = buf[off + j, pl.ds(f * 16, 16)]
                    buf[off + j, pl.ds(f * 16, 16)] = x * wb
            return 0

        lax.fori_loop(0, _CH // 16, grp16, 0)
        pltpu.sync_copy(buf, yt_sh.at[cbuf], add=True)

    pltpu.sync_copy(rs.at[pl.ds(t_lo, _CH)], rs_b)
    pltpu.sync_copy(cs.at[pl.ds(t_lo, _CH)], cs_b)
    pltpu.sync_copy(ws.at[pl.ds(t_lo, _CH)], ws_b)
    pltpu.async_copy(zt.at[rs_b], rowbuf, sem)

    def pair(i, _):
        ka = 2 * i
        kb = ka + 1
        pltpu.sync_copy(rs.at[pl.ds(t_lo + kb * _CH, _CH)], rs_b2)
        pltpu.sync_copy(cs.at[pl.ds(t_lo + kb * _CH, _CH)], cs_b2)
        pltpu.sync_copy(ws.at[pl.ds(t_lo + kb * _CH, _CH)], ws_b2)
        pltpu.async_copy(zt.at[rs_b2], rowbuf2, sem2)
        pltpu.make_async_copy(zt.at[rs_b], rowbuf, sem).wait()
        scale_scatter(rowbuf, ws_b, cs_b, t_lo + ka * _CH)

        @pl.when(i < _NCHS // 2 - 1)
        def _():
            pltpu.sync_copy(rs.at[pl.ds(t_lo + (ka + 2) * _CH, _CH)], rs_b)
            pltpu.sync_copy(cs.at[pl.ds(t_lo + (ka + 2) * _CH, _CH)], cs_b)
            pltpu.sync_copy(ws.at[pl.ds(t_lo + (ka + 2) * _CH, _CH)], ws_b)
            pltpu.async_copy(zt.at[rs_b], rowbuf, sem)

        pltpu.make_async_copy(zt.at[rs_b2], rowbuf2, sem2).wait()
        scale_scatter(rowbuf2, ws_b2, cs_b2, t_lo + kb * _CH)
        return 0

    lax.fori_loop(0, _NCHS // 2, pair, 0)
    plsc.subcore_barrier()
    pltpu.sync_copy(yt_sh.at[pl.ds(sid * _ROWS_T, _ROWS_T)],
                    out.at[cid, pl.ds(sid * _ROWS_T, _ROWS_T)])


def _spmm(zt, rs, cs, ws):
    return pl.kernel(
        _spmm_body,
        out_type=jax.ShapeDtypeStruct((_NC, _NP, _M), jnp.float32),
        mesh=_mesh,
        scratch_types=[
            pltpu.VMEM_SHARED((_NP, _M), jnp.float32),
            pltpu.VMEM((_CH,), jnp.int32),
            pltpu.VMEM((_CH,), jnp.int32),
            pltpu.VMEM((_CH,), jnp.float32),
            pltpu.VMEM((_CH, _M), jnp.float32),
            pltpu.VMEM((_CH,), jnp.int32),
            pltpu.VMEM((_CH,), jnp.int32),
            pltpu.VMEM((_CH,), jnp.float32),
            pltpu.VMEM((_CH, _M), jnp.float32),
            pltpu.SemaphoreType.DMA,
            pltpu.SemaphoreType.DMA,
        ],
        compiler_params=pltpu.CompilerParams(needs_layout_passes=False, use_tc_tiling_on_sc=False),
    )(zt, rs, cs, ws)


def _mv_body(up, rr, cc, ww, idx5, upn,
             av_sh, up_loc, u_loc, av_loc, zb40, r_b, c_b, w_b, idx_loc,
             red16, sem):
    cid = lax.axis_index("c")
    sid = lax.axis_index("s")
    wid = sid * _NC + cid
    lane = lax.broadcasted_iota(jnp.int32, (16,), 0)
    pltpu.sync_copy(up, up_loc)
    pltpu.sync_copy(idx5, idx_loc)

    def addu(i, acc):
        u = up_loc[0, i] + up_loc[1, i]
        u_loc[i] = u
        return acc + u * u

    acc = lax.fori_loop(0, _NP // 16, addu, jnp.zeros((16,), jnp.float32),
                        unroll=4)
    dnums = lax.GatherDimensionNumbers(
        offset_dims=(), collapsed_slice_dims=(0,), start_index_map=(0,))
    for sh in (8, 4, 2, 1):
        perm = jnp.bitwise_xor(lane, sh)[:, None]
        acc = acc + lax.gather(acc, perm, dnums, (1,),
                               mode=lax.GatherScatterMode.PROMISE_IN_BOUNDS)
    s2v = acc
    yi = jnp.int32(0x5F3759DF) - lax.shift_right_logical(
        lax.bitcast_convert_type(s2v, jnp.int32), 1)
    y = lax.bitcast_convert_type(yi, jnp.float32)
    for _ in range(3):
        y = y * (1.5 - 0.5 * s2v * y * y)
    inv_v = y

    zero = jnp.zeros((16,), jnp.float32)

    def zav(i, _):
        av_loc[i] = zero
        return 0

    lax.fori_loop(0, _NP // 16, zav, 0)

    def zzb(i, _):
        zb40[i] = zero
        return 0

    lax.fori_loop(0, _ROWS_T // 16, zzb, 0)
    pltpu.sync_copy(zb40, av_sh.at[pl.ds(sid * (_ROWS_T // 16), _ROWS_T // 16)])
    plsc.subcore_barrier()

    t_lo = wid * _EPT
    pltpu.sync_copy(rr.at[pl.ds(t_lo, _MVCH * _NCHM)], r_b)
    pltpu.sync_copy(cc.at[pl.ds(t_lo, _MVCH * _NCHM)], c_b)
    pltpu.sync_copy(ww.at[pl.ds(t_lo, _MVCH * _NCHM)], w_b)

    def grp(j, _):
        off = j * 16
        c16 = c_b[pl.ds(off, 16)]
        w16 = w_b[pl.ds(off, 16)]
        r16 = r_b[pl.ds(off, 16)]
        g = plsc.load_gather(
            u_loc, [lax.shift_right_logical(c16, 4),
                    jnp.bitwise_and(c16, 15)])
        wv = jnp.where(off + lane < _EPT, w16, 0.0)
        val = g * wv * inv_v
        plsc.addupdate_scatter(
            av_loc, [lax.shift_right_logical(r16, 4),
                     jnp.bitwise_and(r16, 15)], val)
        return 0

    lax.fori_loop(0, _MVCH * _NCHM // 16, grp, 0, unroll=2)
    for k in range(5):
        pltpu.sync_copy(av_loc.at[pl.ds(k * 128, 128)],
                        av_sh.at[idx_loc.at[k]], add=True)
    plsc.subcore_barrier()
    pltpu.sync_copy(av_sh.at[pl.ds(sid * (_ROWS_T // 16), _ROWS_T // 16)],
                    upn.at[cid, pl.ds(sid * (_ROWS_T // 16), _ROWS_T // 16)])


def _matvec(up, rr, cc, ww, idx5):
    return pl.kernel(
        _mv_body,
        out_type=jax.ShapeDtypeStruct((_NC, _NP // 16, 16), jnp.float32),
        mesh=_mesh,
        scratch_types=[
            pltpu.VMEM_SHARED((_NP // 16, 16), jnp.float32),
            pltpu.VMEM((_NC, _NP // 16, 16), jnp.float32),
            pltpu.VMEM((_NP // 16, 16), jnp.float32),
            pltpu.VMEM((_NP // 16, 16), jnp.float32),
            pltpu.VMEM((_ROWS_T // 16, 16), jnp.float32),
            pltpu.VMEM((_MVCH * _NCHM,), jnp.int32),
            pltpu.VMEM((_MVCH * _NCHM,), jnp.int32),
            pltpu.VMEM((_MVCH * _NCHM,), jnp.float32),
            pltpu.VMEM((5, 128), jnp.int32),
            pltpu.VMEM((16,), jnp.float32),
            pltpu.SemaphoreType.DMA,
        ],
        compiler_params=pltpu.CompilerParams(needs_layout_passes=False, use_tc_tiling_on_sc=False),
    )(up, rr, cc, ww, idx5)


_BN = 1280


def _omega_body(f_ref, om_ref, o_ref):
    o_ref[...] = lax.dot_general(
        f_ref[...], om_ref[...], (((0,), (1,)), ((), ())),
        preferred_element_type=jnp.float32)


def _omega(fpad, Om):
    return pl.pallas_call(
        _omega_body,
        grid=(_NP // _BN,),
        in_specs=[pl.BlockSpec((256, _BN), lambda i: (0, i)),
                  pl.BlockSpec((_M, 256), lambda i: (0, 0))],
        out_specs=pl.BlockSpec((_BN, _M), lambda i: (i, 0)),
        out_shape=jax.ShapeDtypeStruct((_NP, _M), jnp.float32),
    )(fpad, Om)


def _pairsum_body(y_ref, o_ref):
    o_ref[...] = y_ref[0] + y_ref[1]


def _pairsum(yp):
    return pl.pallas_call(
        _pairsum_body,
        grid=(_NP // _BN,),
        in_specs=[pl.BlockSpec((2, _BN, _M), lambda i: (0, i, 0))],
        out_specs=pl.BlockSpec((_BN, _M), lambda i: (i, 0)),
        out_shape=jax.ShapeDtypeStruct((_NP, _M), jnp.float32),
    )(yp)


def _first_body(b_ref, w_ref, o_ref):
    xn = jnp.maximum(b_ref[...], 0.0)
    o_ref[...] = lax.dot_general(
        xn, w_ref[...], (((1,), (0,)), ((), ())),
        preferred_element_type=jnp.float32)


def _first(bT, Wp):
    return pl.pallas_call(
        _first_body,
        grid=(_NP // _BN,),
        in_specs=[pl.BlockSpec((_BN, _M), lambda i: (i, 0)),
                  pl.BlockSpec((_M, _M), lambda i: (0, 0))],
        out_specs=pl.BlockSpec((_BN, _M), lambda i: (i, 0)),
        out_shape=jax.ShapeDtypeStruct((_NP, _M), jnp.float32),
    )(bT, Wp)


def _step_body(y_ref, b_ref, w_ref, o_ref):
    xn = jnp.maximum(y_ref[0] + y_ref[1] + b_ref[...], 0.0)
    o_ref[...] = lax.dot_general(
        xn, w_ref[...], (((1,), (0,)), ((), ())),
        preferred_element_type=jnp.float32)


def _step(yp, bT, Wp):
    return pl.pallas_call(
        _step_body,
        grid=(_NP // _BN,),
        in_specs=[pl.BlockSpec((2, _BN, _M), lambda i: (0, i, 0)),
                  pl.BlockSpec((_BN, _M), lambda i: (i, 0)),
                  pl.BlockSpec((_M, _M), lambda i: (0, 0))],
        out_specs=pl.BlockSpec((_BN, _M), lambda i: (i, 0)),
        out_shape=jax.ShapeDtypeStruct((_NP, _M), jnp.float32),
    )(yp, bT, Wp)


def _final_body(y_ref, b_ref, v_ref, o_ref):
    xn = jnp.maximum(y_ref[0] + y_ref[1] + b_ref[...], 0.0)
    s = jnp.sum(xn * xn, axis=1, keepdims=True)
    nrm = jnp.maximum(jnp.sqrt(s), 1e-12)
    o_ref[...] = lax.dot_general(
        xn / nrm, v_ref[...], (((1,), (1,)), ((), ())),
        preferred_element_type=jnp.float32)


def _final(yp, bT, Vw):
    return pl.pallas_call(
        _final_body,
        grid=(_NP // _BN,),
        in_specs=[pl.BlockSpec((2, _BN, _M), lambda i: (0, i, 0)),
                  pl.BlockSpec((_BN, _M), lambda i: (i, 0)),
                  pl.BlockSpec((64, _M), lambda i: (0, 0))],
        out_specs=pl.BlockSpec((_BN, 64), lambda i: (i, 0)),
        out_shape=jax.ShapeDtypeStruct((_NP, 64), jnp.float32),
    )(yp, bT, Vw)


def _proj_body(w_ref, u_ref, o_ref):
    u = u_ref[0:1, :] + u_ref[1:2, :]
    s2 = jnp.sum(u * u)
    arho = jnp.sqrt(s2) + 1e-12
    v = _KAPPA / arho
    A = w_ref[...]
    absA = jnp.abs(A)
    row_sum = jnp.sum(absA, axis=1, keepdims=True)
    hi0 = jnp.max(absA, axis=1, keepdims=True)
    lo0 = jnp.zeros_like(hi0)

    def bis(_, carry):
        lo, hi = carry
        mid = 0.5 * (lo + hi)
        f = jnp.sum(jnp.maximum(absA - mid, 0.0), axis=1, keepdims=True)
        gt = f > v
        return jnp.where(gt, mid, lo), jnp.where(gt, hi, mid)

    lo, hi = lax.fori_loop(0, 60, bis, (lo0, hi0))
    theta = 0.5 * (lo + hi)
    proj = jnp.maximum(absA - theta, 0.0) * jnp.sign(A)
    o_ref[...] = jnp.where(row_sum > v, proj, A)


def _proj(W, up50):
    return pl.pallas_call(
        _proj_body,
        in_specs=[pl.BlockSpec((_M, _M), lambda: (0, 0)),
                  pl.BlockSpec((_NC, _NP), lambda: (0, 0))],
        out_specs=pl.BlockSpec((_M, _M), lambda: (0, 0)),
        out_shape=jax.ShapeDtypeStruct((_M, _M), jnp.float32),
    )(W, up50)


def kernel(features, edge_index, edge_weight, W, Omega_1, V_w):
    f32 = jnp.float32
    row = edge_index[0]
    col = edge_index[1]
    rp = jnp.pad(row, (0, _EPAD - _E))
    cp = jnp.pad(col, (0, _EPAD - _E))
    wp = jnp.pad(edge_weight, (0, _EPAD - _E))
    fpad = jnp.pad(features, ((0, 0), (0, _NP - _N)))
    idx5 = jnp.arange(_NP // 16, dtype=jnp.int32).reshape(5, 128)

    zt0 = _omega(fpad, Omega_1)
    bpair = _spmm(zt0, rp, cp, wp)
    bT = _pairsum(bpair)

    v0 = jnp.concatenate([jnp.full((_N,), 1.0 / np.sqrt(_N), f32),
                          jnp.zeros((_NP - _N,), f32)])
    up0 = jnp.stack([v0, jnp.zeros((_NP,), f32)]).reshape(_NC, _NP // 16, 16)

    def mvstep(up, _):
        return _matvec(up, rp, cp, wp, idx5), None

    up50, _ = lax.scan(mvstep, up0, None, length=_PITERS)
    Wp = _proj(W, up50.reshape(_NC, _NP))

    WpT = Wp.T
    zt = _first(bT, WpT)

    def fpstep(zt, _):
        return _step(_spmm(zt, rp, cp, wp), bT, WpT), None

    zt, _ = lax.scan(fpstep, zt, None, length=_NITER - 2)
    outp = _final(_spmm(zt, rp, cp, wp), bT, V_w)
    return outp[:_N]


# speedup vs baseline: 1.2334x; 1.0001x over previous
import jax
import jax.numpy as jnp
import numpy as np
from jax import lax
from jax.experimental import pallas as pl
from jax.experimental.pallas import tpu as pltpu
from jax.experimental.pallas import tpu_sc as plsc

_N = 10000
_NP = 10240
_M = 128
_E = 160000
_EPAD = 161024
_KAPPA = 0.9
_NITER = 15
_PITERS = 50
_NC = 2
_NS = 16
_NW = _NC * _NS
_EPT = _E // _NW
_CH = 128
_NCHS = 40
_MVCH = 512
_NCHM = 10
_ROWS_T = _NP // _NS

_mesh = plsc.VectorSubcoreMesh(core_axis_name="c", subcore_axis_name="s")


def _spmm_body(zt, rs, cs, ws, out, yt_sh, rs_b, cs_b, ws_b, rowbuf,
               rs_b2, cs_b2, ws_b2, rowbuf2, sem, sem2):
    cid = lax.axis_index("c")
    sid = lax.axis_index("s")
    wid = sid * _NC + cid
    zero = jnp.zeros((16,), jnp.float32)

    def zb(i, _):
        for f in range(8):
            rowbuf[i, pl.ds(f * 16, 16)] = zero
        return 0

    lax.fori_loop(0, _CH, zb, 0)
    for k in range(_ROWS_T // _CH):
        pltpu.sync_copy(rowbuf, yt_sh.at[pl.ds(sid * _ROWS_T + k * _CH, _CH)])
    plsc.subcore_barrier()

    t_lo = wid * _EPT
    t_hi = t_lo + _EPT
    lane = lax.broadcasted_iota(jnp.int32, (16,), 0)

    def scale_scatter(buf, wbuf, cbuf, base):
        def grp16(g, _):
            off = g * 16
            w16 = wbuf[pl.ds(off, 16)]
            wv16 = jnp.where(base + off + lane < t_hi, w16, 0.0)
            for j in range(16):
                wb = jnp.full((16,), wv16[j], jnp.float32)
                for f in range(8):
                    x = buf[off + j, pl.ds(f * 16, 16)]
                    buf[off + j, pl.ds(f * 16, 16)] = x * wb
            return 0

        lax.fori_loop(0, _CH // 16, grp16, 0)
        pltpu.sync_copy(buf, yt_sh.at[cbuf], add=True)

    pltpu.sync_copy(rs.at[pl.ds(t_lo, _CH)], rs_b)
    pltpu.sync_copy(cs.at[pl.ds(t_lo, _CH)], cs_b)
    pltpu.sync_copy(ws.at[pl.ds(t_lo, _CH)], ws_b)
    pltpu.async_copy(zt.at[rs_b], rowbuf, sem)

    def pair(i, _):
        ka = 2 * i
        kb = ka + 1
        pltpu.sync_copy(rs.at[pl.ds(t_lo + kb * _CH, _CH)], rs_b2)
        pltpu.sync_copy(cs.at[pl.ds(t_lo + kb * _CH, _CH)], cs_b2)
        pltpu.sync_copy(ws.at[pl.ds(t_lo + kb * _CH, _CH)], ws_b2)
        pltpu.async_copy(zt.at[rs_b2], rowbuf2, sem2)
        pltpu.make_async_copy(zt.at[rs_b], rowbuf, sem).wait()
        scale_scatter(rowbuf, ws_b, cs_b, t_lo + ka * _CH)

        @pl.when(i < _NCHS // 2 - 1)
        def _():
            pltpu.sync_copy(rs.at[pl.ds(t_lo + (ka + 2) * _CH, _CH)], rs_b)
            pltpu.sync_copy(cs.at[pl.ds(t_lo + (ka + 2) * _CH, _CH)], cs_b)
            pltpu.sync_copy(ws.at[pl.ds(t_lo + (ka + 2) * _CH, _CH)], ws_b)
            pltpu.async_copy(zt.at[rs_b], rowbuf, sem)

        pltpu.make_async_copy(zt.at[rs_b2], rowbuf2, sem2).wait()
        scale_scatter(rowbuf2, ws_b2, cs_b2, t_lo + kb * _CH)
        return 0

    lax.fori_loop(0, _NCHS // 2, pair, 0)
    plsc.subcore_barrier()
    pltpu.sync_copy(yt_sh.at[pl.ds(sid * _ROWS_T, _ROWS_T)],
                    out.at[cid, pl.ds(sid * _ROWS_T, _ROWS_T)])


def _spmm(zt, rs, cs, ws):
    return pl.kernel(
        _spmm_body,
        out_type=jax.ShapeDtypeStruct((_NC, _NP, _M), jnp.float32),
        mesh=_mesh,
        scratch_types=[
            pltpu.VMEM_SHARED((_NP, _M), jnp.float32),
            pltpu.VMEM((_CH,), jnp.int32),
            pltpu.VMEM((_CH,), jnp.int32),
            pltpu.VMEM((_CH,), jnp.float32),
            pltpu.VMEM((_CH, _M), jnp.float32),
            pltpu.VMEM((_CH,), jnp.int32),
            pltpu.VMEM((_CH,), jnp.int32),
            pltpu.VMEM((_CH,), jnp.float32),
            pltpu.VMEM((_CH, _M), jnp.float32),
            pltpu.SemaphoreType.DMA,
            pltpu.SemaphoreType.DMA,
        ],
        compiler_params=pltpu.CompilerParams(needs_layout_passes=False, use_tc_tiling_on_sc=False),
    )(zt, rs, cs, ws)


def _mv_body(up, rr, cc, ww, idx5, upn,
             av_sh, up_loc, u_loc, av_loc, zb40, r_b, c_b, w_b, idx_loc,
             sem):
    cid = lax.axis_index("c")
    sid = lax.axis_index("s")
    wid = sid * _NC + cid
    lane = lax.broadcasted_iota(jnp.int32, (16,), 0)
    pltpu.sync_copy(up, up_loc)
    pltpu.sync_copy(idx5, idx_loc)

    def addu(i, acc):
        u = up_loc[0, i] + up_loc[1, i]
        u_loc[i] = u
        return acc + u * u

    acc = lax.fori_loop(0, _NP // 16, addu, jnp.zeros((16,), jnp.float32),
                        unroll=4)
    dnums = lax.GatherDimensionNumbers(
        offset_dims=(), collapsed_slice_dims=(0,), start_index_map=(0,))
    for sh in (8, 4, 2, 1):
        perm = jnp.bitwise_xor(lane, sh)[:, None]
        acc = acc + lax.gather(acc, perm, dnums, (1,),
                               mode=lax.GatherScatterMode.PROMISE_IN_BOUNDS)
    s2v = acc
    yi = jnp.int32(0x5F3759DF) - lax.shift_right_logical(
        lax.bitcast_convert_type(s2v, jnp.int32), 1)
    y = lax.bitcast_convert_type(yi, jnp.float32)
    for _ in range(3):
        y = y * (1.5 - 0.5 * s2v * y * y)
    inv_v = y

    zero = jnp.zeros((16,), jnp.float32)

    def zav(i, _):
        av_loc[i] = zero
        return 0

    lax.fori_loop(0, _NP // 16, zav, 0)

    def zzb(i, _):
        zb40[i] = zero
        return 0

    lax.fori_loop(0, _ROWS_T // 16, zzb, 0)
    pltpu.sync_copy(zb40, av_sh.at[pl.ds(sid * (_ROWS_T // 16), _ROWS_T // 16)])
    plsc.subcore_barrier()

    t_lo = wid * _EPT
    pltpu.sync_copy(rr.at[pl.ds(t_lo, _MVCH * _NCHM)], r_b)
    pltpu.sync_copy(cc.at[pl.ds(t_lo, _MVCH * _NCHM)], c_b)
    pltpu.sync_copy(ww.at[pl.ds(t_lo, _MVCH * _NCHM)], w_b)

    def grp(j, _):
        off = j * 16
        c16 = c_b[pl.ds(off, 16)]
        w16 = w_b[pl.ds(off, 16)]
        r16 = r_b[pl.ds(off, 16)]
        g = plsc.load_gather(
            u_loc, [lax.shift_right_logical(c16, 4),
                    jnp.bitwise_and(c16, 15)])
        wv = jnp.where(off + lane < _EPT, w16, 0.0)
        val = g * wv * inv_v
        plsc.addupdate_scatter(
            av_loc, [lax.shift_right_logical(r16, 4),
                     jnp.bitwise_and(r16, 15)], val)
        return 0

    lax.fori_loop(0, _MVCH * _NCHM // 16, grp, 0, unroll=2)
    for k in range(5):
        pltpu.sync_copy(av_loc.at[pl.ds(k * 128, 128)],
                        av_sh.at[idx_loc.at[k]], add=True)
    plsc.subcore_barrier()
    pltpu.sync_copy(av_sh.at[pl.ds(sid * (_ROWS_T // 16), _ROWS_T // 16)],
                    upn.at[cid, pl.ds(sid * (_ROWS_T // 16), _ROWS_T // 16)])


def _matvec(up, rr, cc, ww, idx5):
    return pl.kernel(
        _mv_body,
        out_type=jax.ShapeDtypeStruct((_NC, _NP // 16, 16), jnp.float32),
        mesh=_mesh,
        scratch_types=[
            pltpu.VMEM_SHARED((_NP // 16, 16), jnp.float32),
            pltpu.VMEM((_NC, _NP // 16, 16), jnp.float32),
            pltpu.VMEM((_NP // 16, 16), jnp.float32),
            pltpu.VMEM((_NP // 16, 16), jnp.float32),
            pltpu.VMEM((_ROWS_T // 16, 16), jnp.float32),
            pltpu.VMEM((_MVCH * _NCHM,), jnp.int32),
            pltpu.VMEM((_MVCH * _NCHM,), jnp.int32),
            pltpu.VMEM((_MVCH * _NCHM,), jnp.float32),
            pltpu.VMEM((5, 128), jnp.int32),
            pltpu.SemaphoreType.DMA,
        ],
        compiler_params=pltpu.CompilerParams(needs_layout_passes=False, use_tc_tiling_on_sc=False),
    )(up, rr, cc, ww, idx5)


_BN = 1280


def _omega_body(f_ref, om_ref, o_ref):
    o_ref[...] = lax.dot_general(
        f_ref[...], om_ref[...], (((0,), (1,)), ((), ())),
        preferred_element_type=jnp.float32)


def _omega(fpad, Om):
    return pl.pallas_call(
        _omega_body,
        grid=(_NP // _BN,),
        in_specs=[pl.BlockSpec((256, _BN), lambda i: (0, i)),
                  pl.BlockSpec((_M, 256), lambda i: (0, 0))],
        out_specs=pl.BlockSpec((_BN, _M), lambda i: (i, 0)),
        out_shape=jax.ShapeDtypeStruct((_NP, _M), jnp.float32),
    )(fpad, Om)


def _pairsum_body(y_ref, o_ref):
    o_ref[...] = y_ref[0] + y_ref[1]


def _pairsum(yp):
    return pl.pallas_call(
        _pairsum_body,
        grid=(_NP // _BN,),
        in_specs=[pl.BlockSpec((2, _BN, _M), lambda i: (0, i, 0))],
        out_specs=pl.BlockSpec((_BN, _M), lambda i: (i, 0)),
        out_shape=jax.ShapeDtypeStruct((_NP, _M), jnp.float32),
    )(yp)


def _first_body(b_ref, w_ref, o_ref):
    xn = jnp.maximum(b_ref[...], 0.0)
    o_ref[...] = lax.dot_general(
        xn, w_ref[...], (((1,), (0,)), ((), ())),
        preferred_element_type=jnp.float32)


def _first(bT, Wp):
    return pl.pallas_call(
        _first_body,
        grid=(_NP // _BN,),
        in_specs=[pl.BlockSpec((_BN, _M), lambda i: (i, 0)),
                  pl.BlockSpec((_M, _M), lambda i: (0, 0))],
        out_specs=pl.BlockSpec((_BN, _M), lambda i: (i, 0)),
        out_shape=jax.ShapeDtypeStruct((_NP, _M), jnp.float32),
    )(bT, Wp)


def _step_body(y_ref, b_ref, w_ref, o_ref):
    xn = jnp.maximum(y_ref[0] + y_ref[1] + b_ref[...], 0.0)
    o_ref[...] = lax.dot_general(
        xn, w_ref[...], (((1,), (0,)), ((), ())),
        preferred_element_type=jnp.float32)


def _step(yp, bT, Wp):
    return pl.pallas_call(
        _step_body,
        grid=(_NP // _BN,),
        in_specs=[pl.BlockSpec((2, _BN, _M), lambda i: (0, i, 0)),
                  pl.BlockSpec((_BN, _M), lambda i: (i, 0)),
                  pl.BlockSpec((_M, _M), lambda i: (0, 0))],
        out_specs=pl.BlockSpec((_BN, _M), lambda i: (i, 0)),
        out_shape=jax.ShapeDtypeStruct((_NP, _M), jnp.float32),
    )(yp, bT, Wp)


def _final_body(y_ref, b_ref, v_ref, o_ref):
    xn = jnp.maximum(y_ref[0] + y_ref[1] + b_ref[...], 0.0)
    s = jnp.sum(xn * xn, axis=1, keepdims=True)
    nrm = jnp.maximum(jnp.sqrt(s), 1e-12)
    o_ref[...] = lax.dot_general(
        xn / nrm, v_ref[...], (((1,), (1,)), ((), ())),
        preferred_element_type=jnp.float32)


def _final(yp, bT, Vw):
    return pl.pallas_call(
        _final_body,
        grid=(_NP // _BN,),
        in_specs=[pl.BlockSpec((2, _BN, _M), lambda i: (0, i, 0)),
                  pl.BlockSpec((_BN, _M), lambda i: (i, 0)),
                  pl.BlockSpec((64, _M), lambda i: (0, 0))],
        out_specs=pl.BlockSpec((_BN, 64), lambda i: (i, 0)),
        out_shape=jax.ShapeDtypeStruct((_NP, 64), jnp.float32),
    )(yp, bT, Vw)


def _proj_body(w_ref, u_ref, o_ref):
    u = u_ref[0:1, :] + u_ref[1:2, :]
    s2 = jnp.sum(u * u)
    arho = jnp.sqrt(s2) + 1e-12
    v = _KAPPA / arho
    A = w_ref[...]
    absA = jnp.abs(A)
    row_sum = jnp.sum(absA, axis=1, keepdims=True)
    hi0 = jnp.max(absA, axis=1, keepdims=True)
    lo0 = jnp.zeros_like(hi0)

    def bis(_, carry):
        lo, hi = carry
        mid = 0.5 * (lo + hi)
        f = jnp.sum(jnp.maximum(absA - mid, 0.0), axis=1, keepdims=True)
        gt = f > v
        return jnp.where(gt, mid, lo), jnp.where(gt, hi, mid)

    lo, hi = lax.fori_loop(0, 60, bis, (lo0, hi0))
    theta = 0.5 * (lo + hi)
    proj = jnp.maximum(absA - theta, 0.0) * jnp.sign(A)
    o_ref[...] = jnp.where(row_sum > v, proj, A)


def _proj(W, up50):
    return pl.pallas_call(
        _proj_body,
        in_specs=[pl.BlockSpec((_M, _M), lambda: (0, 0)),
                  pl.BlockSpec((_NC, _NP), lambda: (0, 0))],
        out_specs=pl.BlockSpec((_M, _M), lambda: (0, 0)),
        out_shape=jax.ShapeDtypeStruct((_M, _M), jnp.float32),
    )(W, up50)


def kernel(features, edge_index, edge_weight, W, Omega_1, V_w):
    f32 = jnp.float32
    row = edge_index[0]
    col = edge_index[1]
    rp = jnp.pad(row, (0, _EPAD - _E))
    cp = jnp.pad(col, (0, _EPAD - _E))
    wp = jnp.pad(edge_weight, (0, _EPAD - _E))
    fpad = jnp.pad(features, ((0, 0), (0, _NP - _N)))
    idx5 = jnp.arange(_NP // 16, dtype=jnp.int32).reshape(5, 128)

    zt0 = _omega(fpad, Omega_1)
    bpair = _spmm(zt0, rp, cp, wp)
    bT = _pairsum(bpair)

    v0 = jnp.concatenate([jnp.full((_N,), 1.0 / np.sqrt(_N), f32),
                          jnp.zeros((_NP - _N,), f32)])
    up0 = jnp.stack([v0, jnp.zeros((_NP,), f32)]).reshape(_NC, _NP // 16, 16)

    def mvstep(up, _):
        return _matvec(up, rp, cp, wp, idx5), None

    up50, _ = lax.scan(mvstep, up0, None, length=_PITERS)
    Wp = _proj(W, up50.reshape(_NC, _NP))

    WpT = Wp.T
    zt = _first(bT, WpT)

    def fpstep(zt, _):
        return _step(_spmm(zt, rp, cp, wp), bT, WpT), None

    zt, _ = lax.scan(fpstep, zt, None, length=_NITER - 2)
    outp = _final(_spmm(zt, rp, cp, wp), bT, V_w)
    return outp[:_N]
